# baseline (speedup 1.0000x reference)
import functools

import jax
import jax.numpy as jnp
from jax import lax
from jax.experimental import pallas as pl
from jax.experimental.pallas import tpu as pltpu
from jax.experimental.pallas import tpu_sc as plsc

NUM_CORES = 2
NUM_SUBCORES = 16
NUM_WORKERS = NUM_CORES * NUM_SUBCORES
LANES = 16
CHUNK = 128


@jax.jit
def _score_pairs(user_pairs, item_pairs, user_table, item_table):
    batch = user_pairs.shape[0] // 2
    dim = user_table.shape[1]
    b_per_w = batch // NUM_WORKERS
    n_chunks = b_per_w // CHUNK
    n_groups = b_per_w // LANES

    mesh = plsc.VectorSubcoreMesh(
        core_axis_name="c", subcore_axis_name="s",
        num_cores=NUM_CORES, num_subcores=NUM_SUBCORES)

    @functools.partial(
        pl.kernel,
        out_type=jax.ShapeDtypeStruct((batch,), jnp.float32),
        mesh=mesh,
        compiler_params=pltpu.CompilerParams(
            needs_layout_passes=False, use_tc_tiling_on_sc=False),
        scratch_types=[
            pltpu.VMEM((2 * b_per_w,), jnp.int32),
            pltpu.VMEM((2 * b_per_w,), jnp.int32),
            pltpu.VMEM((2 * b_per_w, dim), jnp.float32),
            pltpu.VMEM((2 * b_per_w, dim), jnp.float32),
            pltpu.VMEM((b_per_w,), jnp.float32),
            pltpu.SemaphoreType.DMA,
        ],
    )
    def run(ui_hbm, ii_hbm, ut_hbm, it_hbm, out_hbm,
            idx_u, idx_i, rows_u, rows_i, out_v, sem):
        wid = lax.axis_index("s") * NUM_CORES + lax.axis_index("c")
        base = wid * b_per_w

        pltpu.sync_copy(ui_hbm.at[pl.ds(2 * base, 2 * b_per_w)], idx_u)
        pltpu.sync_copy(ii_hbm.at[pl.ds(2 * base, 2 * b_per_w)], idx_i)

        copies = []
        for c in range(n_chunks):
            blk = pl.ds(c * 2 * CHUNK, 2 * CHUNK)
            copies.append(pltpu.async_copy(
                ut_hbm.at[idx_u.at[blk]], rows_u.at[blk], sem))
            copies.append(pltpu.async_copy(
                it_hbm.at[idx_i.at[blk]], rows_i.at[blk], sem))
        for cp in copies:
            cp.wait()

        lanes = lax.iota(jnp.int32, LANES)
        for g in range(n_groups):
            j0 = g * LANES
            row0 = (j0 // CHUNK) * 2 * CHUNK + (j0 % CHUNK)
            row = row0 + lanes
            acc = jnp.zeros((LANES,), jnp.float32)
            for d in range(dim):
                col = jnp.full((LANES,), d, jnp.int32)
                u = plsc.load_gather(rows_u, [row, col])
                v = plsc.load_gather(rows_i, [row, col])
                acc = acc + u * v
            out_v[pl.ds(j0, LANES)] = acc

        pltpu.sync_copy(out_v, out_hbm.at[pl.ds(base, b_per_w)])

    return run(user_pairs, item_pairs, user_table, item_table)


def _as_index_pairs(idx):
    idx = idx.astype(jnp.int32)
    pairs = jnp.stack([idx * 2, jnp.zeros_like(idx)], axis=-1)
    return pairs.reshape(-1)


def kernel(user_input, item_input, user_table, item_table):
    return _score_pairs(
        _as_index_pairs(user_input),
        _as_index_pairs(item_input),
        user_table, item_table)


# speedup vs baseline: 6.1537x; 6.1537x over previous
import functools

import jax
import jax.numpy as jnp
from jax import lax
from jax.experimental import pallas as pl
from jax.experimental.pallas import tpu as pltpu
from jax.experimental.pallas import tpu_sc as plsc

NUM_CORES = 2
NUM_SUBCORES = 16
NUM_WORKERS = NUM_CORES * NUM_SUBCORES
LANES = 16


@jax.jit
def _score_pairs(user_pairs, item_pairs, u0, u1, u2, u3, i0, i1, i2, i3):
    batch = user_pairs.shape[0] // 2
    b_per_w = batch // NUM_WORKERS
    n_groups = b_per_w // LANES
    span = 2 * b_per_w

    mesh = plsc.VectorSubcoreMesh(
        core_axis_name="c", subcore_axis_name="s",
        num_cores=NUM_CORES, num_subcores=NUM_SUBCORES)

    @functools.partial(
        pl.kernel,
        out_type=jax.ShapeDtypeStruct((batch,), jnp.float32),
        mesh=mesh,
        compiler_params=pltpu.CompilerParams(
            needs_layout_passes=False, use_tc_tiling_on_sc=False),
        scratch_types=[
            pltpu.VMEM((span,), jnp.int32),
            pltpu.VMEM((span,), jnp.int32),
            pltpu.VMEM((4 * span,), jnp.float32),
            pltpu.VMEM((4 * span,), jnp.float32),
            pltpu.VMEM((b_per_w,), jnp.float32),
            pltpu.SemaphoreType.DMA,
        ],
    )
    def run(up_hbm, ip_hbm, u0h, u1h, u2h, u3h, i0h, i1h, i2h, i3h, out_hbm,
            idx_u, idx_i, cols_u, cols_i, out_v, sem):
        wid = lax.axis_index("s") * NUM_CORES + lax.axis_index("c")
        base = wid * b_per_w

        pltpu.sync_copy(up_hbm.at[pl.ds(2 * base, span)], idx_u)
        pltpu.sync_copy(ip_hbm.at[pl.ds(2 * base, span)], idx_i)

        copies = []
        for d, col in enumerate((u0h, u1h, u2h, u3h)):
            copies.append(pltpu.async_copy(
                col.at[idx_u], cols_u.at[pl.ds(d * span, span)], sem))
        for d, col in enumerate((i0h, i1h, i2h, i3h)):
            copies.append(pltpu.async_copy(
                col.at[idx_i], cols_i.at[pl.ds(d * span, span)], sem))
        for cp in copies:
            cp.wait()

        lanes = lax.iota(jnp.int32, LANES)
        for g in range(n_groups):
            pos = (g * LANES + lanes) * 2
            acc = jnp.zeros((LANES,), jnp.float32)
            for d in range(4):
                u = plsc.load_gather(cols_u, [d * span + pos])
                v = plsc.load_gather(cols_i, [d * span + pos])
                acc = acc + u * v
            out_v[pl.ds(g * LANES, LANES)] = acc

        pltpu.sync_copy(out_v, out_hbm.at[pl.ds(base, b_per_w)])

    return run(user_pairs, item_pairs, u0, u1, u2, u3, i0, i1, i2, i3)


def _as_index_pairs(idx):
    idx = idx.astype(jnp.int32)
    return jnp.stack([idx, jnp.zeros_like(idx)], axis=-1).reshape(-1)


def kernel(user_input, item_input, user_table, item_table):
    return _score_pairs(
        _as_index_pairs(user_input),
        _as_index_pairs(item_input),
        user_table[:, 0], user_table[:, 1], user_table[:, 2], user_table[:, 3],
        item_table[:, 0], item_table[:, 1], item_table[:, 2], item_table[:, 3])
